# Initial kernel scaffold; baseline (speedup 1.0000x reference)
import jax
import jax.numpy as jnp
from jax.experimental import pallas as pl


def kernel(mobility, text, casex, idx, Wg, Wi, Wh, b_lstm, W1, b1, W2, b2, Wfc, bfc):
    raise NotImplementedError("write your pallas kernel here")


# speedup vs baseline: 3.0169x; 3.0169x over previous
import jax
import jax.numpy as jnp
from jax.experimental import pallas as pl
from jax.experimental.pallas import tpu as pltpu


def _dinv_body(mob_ref, dinv_ref):
    rs = jnp.sum(mob_ref[...], axis=1, keepdims=True)
    dinv_ref[...] = jax.lax.rsqrt(rs + 2.0 + 1e-8)


def _make_lstm_body(nz, nb, hid, ch):
    nchunks = nz // ch

    def body(x_ref, wi_ref, wh_ref, b_ref, cflat_ref, dinv_ref,
             l_ref, cs_ref, xp_scr):
        cs_ref[...] = cflat_ref[...] * dinv_ref[...]
        wh = wh_ref[...]

        def chunk(cidx, carry):
            xp_scr[...] = jnp.dot(
                x_ref[pl.ds(cidx * ch * nb, ch * nb), :], wi_ref[...],
                preferred_element_type=jnp.float32) + b_ref[...]

            def step(t, carry):
                h, c = carry
                xp = xp_scr[pl.ds(t * nb, nb), :]
                gates = xp + jnp.dot(h, wh,
                                     preferred_element_type=jnp.float32)
                i_, f_, g_, o_ = jnp.split(gates, 4, axis=1)
                c = (jax.nn.sigmoid(f_) * c
                     + jax.nn.sigmoid(i_) * jnp.tanh(g_))
                h = jax.nn.sigmoid(o_) * jnp.tanh(c)
                l_ref[pl.ds((cidx * ch + t) * nb, nb), :] = h
                return (h, c)

            return jax.lax.fori_loop(0, ch, step, carry)

        h0 = jnp.zeros((nb, hid), jnp.float32)
        jax.lax.fori_loop(0, nchunks, chunk, (h0, h0))
    return body


def _make_pass1_body(rb, n):
    def body(mob_ref, text_ref, textt_ref, wg_ref, dinv_ref, cs_ref,
             l_ref, w1_ref, b1_ref, h1s_ref):
        i = pl.program_id(0)
        tg = jnp.dot(text_ref[...], wg_ref[...],
                     preferred_element_type=jnp.float32)
        s = jnp.dot(tg, textt_ref[...], preferred_element_type=jnp.float32)
        s = jnp.maximum(s, 0.0)
        m = jnp.max(s, axis=1, keepdims=True)
        e = jnp.exp(s - m)
        p = e / jnp.sum(e, axis=1, keepdims=True)
        rows = jax.lax.broadcasted_iota(jnp.int32, (rb, n), 0) + i * rb
        cols = jax.lax.broadcasted_iota(jnp.int32, (rb, n), 1)
        a = p + mob_ref[...] + jnp.where(rows == cols, 1.0, 0.0)
        y1 = dinv_ref[...] * jnp.dot(a, cs_ref[...],
                                     preferred_element_type=jnp.float32)
        h1 = jnp.maximum(
            jnp.dot(y1, w1_ref[...], preferred_element_type=jnp.float32)
            + b1_ref[...], 0.0) + l_ref[...]
        h1s_ref[...] = dinv_ref[...] * h1
    return body


def _make_pass2_body(rb, n):
    def body(mob_ref, text_ref, textt_ref, wg_ref, dinv_ref, h1s_ref,
             cflat_ref, w2_ref, b2_ref, z_ref):
        i = pl.program_id(0)
        tg = jnp.dot(text_ref[...], wg_ref[...],
                     preferred_element_type=jnp.float32)
        s = jnp.dot(tg, textt_ref[...], preferred_element_type=jnp.float32)
        s = jnp.maximum(s, 0.0)
        m = jnp.max(s, axis=1, keepdims=True)
        e = jnp.exp(s - m)
        p = e / jnp.sum(e, axis=1, keepdims=True)
        rows = jax.lax.broadcasted_iota(jnp.int32, (rb, n), 0) + i * rb
        cols = jax.lax.broadcasted_iota(jnp.int32, (rb, n), 1)
        a = p + mob_ref[...] + jnp.where(rows == cols, 1.0, 0.0)
        y2 = dinv_ref[...] * jnp.dot(a, h1s_ref[...],
                                     preferred_element_type=jnp.float32)
        h2 = jnp.dot(y2, w2_ref[...],
                     preferred_element_type=jnp.float32) + b2_ref[...]
        contrib = jax.lax.dot_general(
            cflat_ref[...], h2, (((0,), (0,)), ((), ())),
            preferred_element_type=jnp.float32)

        @pl.when(i == 0)
        def _():
            z_ref[...] = jnp.zeros_like(z_ref)

        z_ref[...] += contrib
    return body


def _fc_body(zf_ref, wfc_ref, bfc_ref, out_ref):
    k = pl.program_id(0)

    @pl.when(k == 0)
    def _():
        out_ref[...] = jnp.broadcast_to(bfc_ref[...], out_ref.shape)

    out_ref[...] += jnp.dot(zf_ref[...], wfc_ref[...],
                            preferred_element_type=jnp.float32)


def kernel(mobility, text, casex, idx, Wg, Wi, Wh, b_lstm,
           W1, b1, W2, b2, Wfc, bfc):
    f32 = jnp.float32
    mobility = mobility.astype(f32)
    text = text.astype(f32)
    casex = casex.astype(f32)
    N = mobility.shape[0]
    B, _, XD = casex.shape
    HID = Wh.shape[0]
    G4 = Wh.shape[1]
    OUT = W2.shape[1]
    FEAT = Wg.shape[0]
    YTOT = Wfc.shape[1]

    RB = 200 if N % 200 == 0 else N

    casexT = jnp.transpose(casex, (1, 0, 2))
    Cflat = casexT.reshape(N, B * XD)
    xflat = casexT.reshape(N * B, XD)
    textT = jnp.transpose(text)
    eyeB = jnp.eye(B, dtype=f32)
    W1bd = jnp.kron(eyeB, W1)
    W2bd = jnp.kron(eyeB, W2)
    b1t = jnp.tile(b1, B).reshape(1, B * HID)
    b2t = jnp.tile(b2, B).reshape(1, B * OUT)
    blstm2 = b_lstm.reshape(1, G4)
    bfc2 = bfc.reshape(1, YTOT)

    dinv = pl.pallas_call(
        _dinv_body,
        grid=(N // RB,),
        in_specs=[pl.BlockSpec((RB, N), lambda i: (i, 0))],
        out_specs=pl.BlockSpec((RB, 1), lambda i: (i, 0)),
        out_shape=jax.ShapeDtypeStruct((N, 1), f32),
    )(mobility)

    CH = 500 if N % 500 == 0 else N
    lstm_flat, Cs = pl.pallas_call(
        _make_lstm_body(N, B, HID, CH),
        out_shape=[jax.ShapeDtypeStruct((N * B, HID), f32),
                   jax.ShapeDtypeStruct((N, B * XD), f32)],
        scratch_shapes=[pltpu.VMEM((CH * B, G4), f32)],
    )(xflat, Wi, Wh, blstm2, Cflat, dinv)
    L = lstm_flat.reshape(N, B * HID)

    full = lambda shape: pl.BlockSpec(shape, lambda i: tuple(0 for _ in shape))
    h1s = pl.pallas_call(
        _make_pass1_body(RB, N),
        grid=(N // RB,),
        in_specs=[
            pl.BlockSpec((RB, N), lambda i: (i, 0)),
            pl.BlockSpec((RB, FEAT), lambda i: (i, 0)),
            full((FEAT, N)),
            full((FEAT, FEAT)),
            pl.BlockSpec((RB, 1), lambda i: (i, 0)),
            full((N, B * XD)),
            pl.BlockSpec((RB, B * HID), lambda i: (i, 0)),
            full((B * XD, B * HID)),
            full((1, B * HID)),
        ],
        out_specs=pl.BlockSpec((RB, B * HID), lambda i: (i, 0)),
        out_shape=jax.ShapeDtypeStruct((N, B * HID), f32),
    )(mobility, text, textT, Wg, dinv, Cs, L, W1bd, b1t)

    z8 = pl.pallas_call(
        _make_pass2_body(RB, N),
        grid=(N // RB,),
        in_specs=[
            pl.BlockSpec((RB, N), lambda i: (i, 0)),
            pl.BlockSpec((RB, FEAT), lambda i: (i, 0)),
            full((FEAT, N)),
            full((FEAT, FEAT)),
            pl.BlockSpec((RB, 1), lambda i: (i, 0)),
            full((N, B * HID)),
            pl.BlockSpec((RB, B * XD), lambda i: (i, 0)),
            full((B * HID, B * OUT)),
            full((1, B * OUT)),
        ],
        out_specs=full((B * XD, B * OUT)),
        out_shape=jax.ShapeDtypeStruct((B * XD, B * OUT), f32),
    )(mobility, text, textT, Wg, dinv, h1s, Cflat, W2bd, b2t)

    z8 = z8 / N
    zf = jnp.stack([
        jax.lax.slice(z8, (b * XD, b * OUT), ((b + 1) * XD, (b + 1) * OUT))
        for b in range(B)
    ]).reshape(B, XD * OUT)

    K = XD * OUT
    KB = 128 if K % 128 == 0 else K
    out = pl.pallas_call(
        _fc_body,
        grid=(K // KB,),
        in_specs=[
            pl.BlockSpec((B, KB), lambda k: (0, k)),
            pl.BlockSpec((KB, YTOT), lambda k: (k, 0)),
            pl.BlockSpec((1, YTOT), lambda k: (0, 0)),
        ],
        out_specs=pl.BlockSpec((B, YTOT), lambda k: (0, 0)),
        out_shape=jax.ShapeDtypeStruct((B, YTOT), f32),
    )(zf, Wfc, bfc2)

    return out.reshape(B, YTOT // N, N)
